# baseline (speedup 1.0000x reference)
import functools

import jax
import jax.numpy as jnp
from jax import lax
from jax.experimental import pallas as pl
from jax.experimental.pallas import tpu as pltpu
from jax.experimental.pallas import tpu_sc as plsc

_NC = 2
_NS = 16
_NW = _NC * _NS
_LANES = 16


def _dist_argmin_body(z_ref, w_ref, w2_ref, ind_ref):
    z = z_ref[...]
    w = w_ref[...]
    w2 = w2_ref[...]
    z2 = jnp.sum(z * z, axis=1, keepdims=True)
    e = jax.lax.dot_general(z.astype(jnp.bfloat16), w.astype(jnp.bfloat16),
                            (((1,), (1,)), ((), ())),
                            preferred_element_type=jnp.float32)
    d = (z2 - 2.0 * e) + w2
    iota_f = jax.lax.broadcasted_iota(jnp.int32, d.shape, 1).astype(jnp.float32)
    m = jnp.min(d, axis=1, keepdims=True)
    ind_f = jnp.min(jnp.where(d == m, iota_f, float(d.shape[1])), axis=1)
    ind_ref[0, 0, :] = ind_f.astype(jnp.int32)


def _dist_argmin(zf, W, w2, tm):
    n, d_model = zf.shape
    k_cb = W.shape[0]
    g = n // tm
    return pl.pallas_call(
        _dist_argmin_body,
        grid=(g,),
        in_specs=[
            pl.BlockSpec((tm, d_model), lambda i: (i, 0)),
            pl.BlockSpec((k_cb, d_model), lambda i: (0, 0)),
            pl.BlockSpec((1, k_cb), lambda i: (0, 0)),
        ],
        out_specs=pl.BlockSpec((1, 1, tm), lambda i: (i, 0, 0)),
        out_shape=jax.ShapeDtypeStruct((g, 1, tm), jnp.int32),
    )(zf, W, w2)


def _sc_lookup(W, ind_flat, zf):
    n, d_model = zf.shape
    rows = n // _NW
    half = (rows // 2 + 7) // 8 * 8
    mesh = plsc.VectorSubcoreMesh(core_axis_name="c", subcore_axis_name="s")

    @functools.partial(
        pl.kernel,
        mesh=mesh,
        out_type=(
            jax.ShapeDtypeStruct((n, d_model), jnp.float32),
            jax.ShapeDtypeStruct((_NW, _LANES), jnp.float32),
        ),
        scratch_types=[
            pltpu.VMEM((rows,), jnp.int32),
            pltpu.VMEM((rows, d_model), jnp.float32),
            pltpu.VMEM((rows, d_model), jnp.float32),
            pltpu.VMEM((_LANES,), jnp.float32),
            pltpu.SemaphoreType.DMA,
        ],
    )
    def sc_kernel(w_hbm, idx_hbm, z_hbm, zq_hbm, part_hbm,
                  idx_v, rows_v, z_v, acc_v, sem):
        wid = lax.axis_index("s") * _NC + lax.axis_index("c")
        base = wid * rows
        pltpu.sync_copy(idx_hbm.at[pl.ds(base, rows)], idx_v)
        cp1 = pltpu.async_copy(w_hbm.at[idx_v.at[pl.ds(0, half)]],
                               rows_v.at[pl.ds(0, half)], sem)
        cp2 = pltpu.async_copy(w_hbm.at[idx_v.at[pl.ds(half, rows - half)]],
                               rows_v.at[pl.ds(half, rows - half)], sem)
        pltpu.sync_copy(z_hbm.at[pl.ds(base, rows)], z_v)
        cp1.wait()
        cp2.wait()

        def body(r, acc):
            for c in range(d_model // _LANES):
                sl = pl.ds(c * _LANES, _LANES)
                zq = rows_v[r, sl]
                zz = z_v[r, sl]
                dlt = zq - zz
                rows_v[r, sl] = zz + dlt
                acc = acc + dlt * dlt
            return acc

        acc = lax.fori_loop(0, rows, body, jnp.zeros((_LANES,), jnp.float32))
        acc_v[...] = acc
        pltpu.sync_copy(rows_v, zq_hbm.at[pl.ds(base, rows)])
        pltpu.sync_copy(acc_v, part_hbm.at[wid])

    return sc_kernel(W, ind_flat, zf)


def kernel(z, W):
    b, t, d_model = z.shape
    n = b * t
    tm = 1152
    zf = z.reshape(n, d_model)
    w2 = jnp.sum(W ** 2, axis=-1).reshape(1, W.shape[0])
    ind3 = _dist_argmin(zf, W, w2, tm)
    ind_flat = ind3.reshape(n)
    zq_st, part = _sc_lookup(W, ind_flat, zf)
    loss = 2.0 * jnp.sum(part) / (n * d_model)
    return zq_st.reshape(b, t, d_model), ind3.reshape(b, t), loss


# speedup vs baseline: 1.9724x; 1.9724x over previous
import jax
import jax.numpy as jnp
from jax.experimental import pallas as pl


def _vq_body(z_ref, w_ref, w2_ref, iota_ref, zq_ref, ind_ref, loss_ref):
    z = z_ref[...]
    w = w_ref[...]
    w2 = w2_ref[...]
    iota_f = iota_ref[...]
    z2 = jnp.sum(z * z, axis=1, keepdims=True)
    e = jax.lax.dot_general(z.astype(jnp.bfloat16), w.astype(jnp.bfloat16),
                            (((1,), (1,)), ((), ())),
                            preferred_element_type=jnp.float32)
    d = (z2 - 2.0 * e) + w2
    m = jnp.min(d, axis=1, keepdims=True)
    ind_f = jnp.min(jnp.where(d == m, iota_f, float(d.shape[1])),
                    axis=1, keepdims=True)
    ind_ref[0, 0, :] = ind_f[:, 0].astype(jnp.int32)
    oh = (iota_f == ind_f).astype(jnp.bfloat16)
    zq = jax.lax.dot_general(oh, w.astype(jnp.bfloat16), (((1,), (0,)), ((), ())),
                             preferred_element_type=jnp.float32)
    diff = zq - z
    zq_ref[...] = z + diff
    part = jnp.broadcast_to(jnp.sum(diff * diff), (128,))

    @pl.when(pl.program_id(0) == 0)
    def _():
        loss_ref[0, 0, :] = part

    @pl.when(pl.program_id(0) != 0)
    def _():
        loss_ref[0, 0, :] += part


def _vq_pallas(zf, W, w2, iota_f, tm, interpret=False):
    n, d_model = zf.shape
    k_cb = W.shape[0]
    g = n // tm
    out_shapes = (
        jax.ShapeDtypeStruct((n, d_model), jnp.float32),
        jax.ShapeDtypeStruct((g, 1, tm), jnp.int32),
        jax.ShapeDtypeStruct((1, 1, 128), jnp.float32),
    )
    return pl.pallas_call(
        _vq_body,
        grid=(g,),
        in_specs=[
            pl.BlockSpec((tm, d_model), lambda i: (i, 0)),
            pl.BlockSpec((k_cb, d_model), lambda i: (0, 0)),
            pl.BlockSpec((1, k_cb), lambda i: (0, 0)),
            pl.BlockSpec((1, k_cb), lambda i: (0, 0)),
        ],
        out_specs=(
            pl.BlockSpec((tm, d_model), lambda i: (i, 0)),
            pl.BlockSpec((1, 1, tm), lambda i: (i, 0, 0)),
            pl.BlockSpec((1, 1, 128), lambda i: (0, 0, 0)),
        ),
        out_shape=out_shapes,
        interpret=interpret,
    )(zf, W, w2, iota_f)


def kernel(z, W):
    b, t, d_model = z.shape
    n = b * t
    tm = 1152
    k_cb = W.shape[0]
    zf = z.reshape(n, d_model)
    w2 = jnp.sum(W ** 2, axis=-1).reshape(1, k_cb)
    iota_f = jax.lax.iota(jnp.float32, k_cb).reshape(1, k_cb)
    zq_st, ind3, part = _vq_pallas(zf, W, w2, iota_f, tm)
    loss = part[0, 0, 0] * (2.0 / (n * d_model))
    return zq_st.reshape(b, t, d_model), ind3.reshape(b, t), loss


# speedup vs baseline: 2.1498x; 1.0900x over previous
import jax
import jax.numpy as jnp
from jax.experimental import pallas as pl


def _vq_body(z_ref, w_ref, iota_ref, zq_ref, ind_ref, loss_ref):
    z = z_ref[...]
    w = w_ref[...]
    iota_f = iota_ref[...]
    w2 = jnp.sum(w * w, axis=1)[None, :]
    z2 = jnp.sum(z * z, axis=1, keepdims=True)
    e = jax.lax.dot_general(z.astype(jnp.bfloat16), w.astype(jnp.bfloat16),
                            (((1,), (1,)), ((), ())),
                            preferred_element_type=jnp.float32)
    d = (z2 - 2.0 * e) + w2
    m = jnp.min(d, axis=1, keepdims=True)
    ind_f = jnp.min(jnp.where(d == m, iota_f, float(d.shape[1])),
                    axis=1, keepdims=True)
    ind_ref[0, 0, :] = ind_f[:, 0].astype(jnp.int32)
    oh = (iota_f == ind_f).astype(jnp.bfloat16)
    zq = jax.lax.dot_general(oh, w.astype(jnp.bfloat16), (((1,), (0,)), ((), ())),
                             preferred_element_type=jnp.float32)
    diff = zq - z
    zq_ref[...] = z + diff
    part = jnp.broadcast_to(jnp.sum(diff * diff), (128,))

    @pl.when(pl.program_id(0) == 0)
    def _():
        loss_ref[0, 0, :] = part

    @pl.when(pl.program_id(0) != 0)
    def _():
        loss_ref[0, 0, :] += part


def _vq_pallas(zf, W, iota_f, tm, interpret=False):
    n, d_model = zf.shape
    k_cb = W.shape[0]
    g = n // tm
    out_shapes = (
        jax.ShapeDtypeStruct((n, d_model), jnp.float32),
        jax.ShapeDtypeStruct((g, 1, tm), jnp.int32),
        jax.ShapeDtypeStruct((1, 1, 128), jnp.float32),
    )
    return pl.pallas_call(
        _vq_body,
        grid=(g,),
        in_specs=[
            pl.BlockSpec((tm, d_model), lambda i: (i, 0)),
            pl.BlockSpec((k_cb, d_model), lambda i: (0, 0)),
            pl.BlockSpec((1, k_cb), lambda i: (0, 0)),
        ],
        out_specs=(
            pl.BlockSpec((tm, d_model), lambda i: (i, 0)),
            pl.BlockSpec((1, 1, tm), lambda i: (i, 0, 0)),
            pl.BlockSpec((1, 1, 128), lambda i: (0, 0, 0)),
        ),
        out_shape=out_shapes,
        interpret=interpret,
    )(zf, W, iota_f)


def kernel(z, W):
    b, t, d_model = z.shape
    n = b * t
    tm = 1152
    k_cb = W.shape[0]
    zf = z.reshape(n, d_model)
    iota_f = jax.lax.iota(jnp.float32, k_cb).reshape(1, k_cb)
    zq_st, ind3, part = _vq_pallas(zf, W, iota_f, tm)
    loss = part[0, 0, 0] * (2.0 / (n * d_model))
    return zq_st.reshape(b, t, d_model), ind3.reshape(b, t), loss


# speedup vs baseline: 2.1997x; 1.0232x over previous
import jax
import jax.numpy as jnp
from jax.experimental import pallas as pl


def _vq_body(z_ref, w_ref, iota_ref, zq_ref, ind_ref, loss_ref):
    z = z_ref[...]
    w = w_ref[...]
    iota_f = iota_ref[...]
    w2 = jnp.sum(w * w, axis=1)[None, :]
    z2 = jnp.sum(z * z, axis=1, keepdims=True)
    e = jax.lax.dot_general(z.astype(jnp.bfloat16), w.astype(jnp.bfloat16),
                            (((1,), (1,)), ((), ())),
                            preferred_element_type=jnp.float32)
    d = (z2 - 2.0 * e) + w2
    m = jnp.min(d, axis=1, keepdims=True)
    ind_f = jnp.min(jnp.where(d == m, iota_f, float(d.shape[1])),
                    axis=1, keepdims=True)
    ind_ref[0, 0, :] = ind_f[:, 0].astype(jnp.int32)
    oh = (iota_f == ind_f).astype(jnp.bfloat16)
    zq = jax.lax.dot_general(oh, w.astype(jnp.bfloat16), (((1,), (0,)), ((), ())),
                             preferred_element_type=jnp.float32)
    diff = zq - z
    zq_ref[...] = z + diff
    part = jnp.broadcast_to(jnp.sum(diff * diff), (128,))

    @pl.when(pl.program_id(0) == 0)
    def _():
        loss_ref[0, 0, :] = part

    @pl.when(pl.program_id(0) != 0)
    def _():
        loss_ref[0, 0, :] += part


def _vq_pallas(zf, W, iota_f, tm, interpret=False):
    n, d_model = zf.shape
    k_cb = W.shape[0]
    g = n // tm
    out_shapes = (
        jax.ShapeDtypeStruct((n, d_model), jnp.float32),
        jax.ShapeDtypeStruct((g, 1, tm), jnp.int32),
        jax.ShapeDtypeStruct((1, 1, 128), jnp.float32),
    )
    return pl.pallas_call(
        _vq_body,
        grid=(g,),
        in_specs=[
            pl.BlockSpec((tm, d_model), lambda i: (i, 0)),
            pl.BlockSpec((k_cb, d_model), lambda i: (0, 0)),
            pl.BlockSpec((1, k_cb), lambda i: (0, 0)),
        ],
        out_specs=(
            pl.BlockSpec((tm, d_model), lambda i: (i, 0)),
            pl.BlockSpec((1, 1, tm), lambda i: (i, 0, 0)),
            pl.BlockSpec((1, 1, 128), lambda i: (0, 0, 0)),
        ),
        out_shape=out_shapes,
        interpret=interpret,
    )(zf, W, iota_f)


def kernel(z, W):
    b, t, d_model = z.shape
    n = b * t
    tm = 2304
    k_cb = W.shape[0]
    zf = z.reshape(n, d_model)
    iota_f = jax.lax.iota(jnp.float32, k_cb).reshape(1, k_cb)
    zq_st, ind3, part = _vq_pallas(zf, W, iota_f, tm)
    loss = part[0, 0, 0] * (2.0 / (n * d_model))
    return zq_st.reshape(b, t, d_model), ind3.reshape(b, t), loss
